# baseline (speedup 1.0000x reference)
import dataclasses
import functools

import jax
import jax.numpy as jnp
from jax import lax
from jax.experimental import pallas as pl
from jax.experimental.pallas import tpu as pltpu
from jax.experimental.pallas import tpu_sc as plsc

N = 10000
E = 160000
IN = 256
H = 8
D = 64
HID = 128

NC = 2
NS = 16
EPW = E // NS
CH = 80
NCHUNK = EPW // CH
NPAD = 10112
NPW = NPAD // NS
HP = H // 2
PD = 2 * D
NB = 10
BLK = N // NB


def _tc1_body(h_ref, W0_ref, al0_ref, ar0_ref, W1_ref, al1_ref, ar1_ref,
              feat_ref, eld_ref, erd_ref):
    hb = h_ref[...]
    for p, (W_ref, al_ref, ar_ref) in enumerate(
            [(W0_ref, al0_ref, ar0_ref), (W1_ref, al1_ref, ar1_ref)]):
        f = jnp.dot(hb, W_ref[...], preferred_element_type=jnp.float32)
        feat_ref[p, :, :] = f
        fh = f.reshape(BLK, H, D)
        el = (fh * al_ref[...][None]).sum(-1)
        er = (fh * ar_ref[...][None]).sum(-1)
        eld_ref[p, :, :] = jnp.concatenate([el, el], axis=1)
        erd_ref[p, :, :] = jnp.concatenate([er, er], axis=1)


def _tc1(h, W0, al0, ar0, W1, al1, ar1):
    return pl.pallas_call(
        _tc1_body,
        grid=(NB,),
        in_specs=[
            pl.BlockSpec((BLK, IN), lambda i: (i, 0)),
            pl.BlockSpec((IN, H * D), lambda i: (0, 0)),
            pl.BlockSpec((H, D), lambda i: (0, 0)),
            pl.BlockSpec((H, D), lambda i: (0, 0)),
            pl.BlockSpec((IN, H * D), lambda i: (0, 0)),
            pl.BlockSpec((H, D), lambda i: (0, 0)),
            pl.BlockSpec((H, D), lambda i: (0, 0)),
        ],
        out_specs=[
            pl.BlockSpec((NC, BLK, H * D), lambda i: (0, i, 0)),
            pl.BlockSpec((NC, BLK, 2 * H), lambda i: (0, i, 0)),
            pl.BlockSpec((NC, BLK, 2 * H), lambda i: (0, i, 0)),
        ],
        out_shape=[
            jax.ShapeDtypeStruct((NC, N, H * D), jnp.float32),
            jax.ShapeDtypeStruct((NC, N, 2 * H), jnp.float32),
            jax.ShapeDtypeStruct((NC, N, 2 * H), jnp.float32),
        ],
    )(h, W0, al0, ar0, W1, al1, ar1)


def _sc_gat_body(featv, eldv, erdv, sd, zacc, zden, accs, dens, exs,
                 acc_s, den_s,
                 sdt0, sdt1, db0, db1, db2, ia0, ia1, ia2, ib0,
                 ga, gb, exc, exb0, exb1, exb2, g0, g1, g2,
                 semg0, semg1, semg2, semx0, semx1, semx2,
                 semc0, semc1, semc2, semt0, semd1):
    c = lax.axis_index("c")
    s = lax.axis_index("s")
    nbase = pl.multiple_of(s * NPW, NPW)
    eoff = c * N
    fbase = c * (N * HP)

    db = [db0, db1, db2]
    ia = [ia0, ia1, ia2]
    exb = [exb0, exb1, exb2]
    g = [g0, g1, g2]
    semg = [semg0, semg1, semg2]
    semx = [semx0, semx1, semx2]
    semc = [semc0, semc1, semc2]

    sd_cs = sd.at[c].at[s]
    exs_cs = exs.at[c].at[s]
    mask8 = jnp.arange(16, dtype=jnp.int32) < 8

    pltpu.sync_copy(zden, den_s.at[pl.ds(nbase, NPW)])
    plsc.subcore_barrier()

    @pl.loop(0, NCHUNK // 5)
    def _(qq):
        pltpu.sync_copy(sd_cs.at[pl.ds(qq * 5, 5)], sdt0)
        for k in range(5):
            i = qq * 5 + k

            def _waits():
                pltpu.make_async_copy(exc.at[pl.ds(0, 8 * CH)],
                                      exs_cs.at[0], semt0).wait()
                pltpu.make_async_copy(ga, den_s.at[db[0]], semc0).wait()

            if k == 0:
                pl.when(qq > 0)(_waits)
            else:
                _waits()

            @pl.loop(0, CH, step=16)
            def _(u):
                srow = sdt0[k, 0, pl.ds(u, 16)]
                drow = sdt0[k, 1, pl.ds(u, 16)]
                ia[0][pl.ds(u, 16)] = srow + eoff
                ib0[pl.ds(u, 16)] = drow + eoff
                db[0][pl.ds(u, 16)] = drow
            pltpu.async_copy(eldv.at[ia[0]], ga, semg0)
            pltpu.async_copy(erdv.at[ib0], gb, semx0)
            pltpu.make_async_copy(eldv.at[ia[0]], ga, semg0).wait()
            pltpu.make_async_copy(erdv.at[ib0], gb, semx0).wait()

            @pl.loop(0, CH)
            def _(r):
                x = ga[r, :] + gb[r, :]
                x = jnp.maximum(x, 0.0) + 0.2 * jnp.minimum(x, 0.0)
                x = jnp.exp(x)
                ga[r, :] = x
                plsc.store_compressed(exc.at[pl.ds(r * 8, 16)], x, mask=mask8)

            pltpu.async_copy(exc.at[pl.ds(0, 8 * CH)], exs_cs.at[i], semt0)
            pltpu.async_copy(ga, den_s.at[db[0]], semc0, add=True)

    pltpu.make_async_copy(exc.at[pl.ds(0, 8 * CH)], exs_cs.at[0], semt0).wait()
    pltpu.make_async_copy(ga, den_s.at[db[0]], semc0).wait()

    plsc.subcore_barrier()
    pltpu.sync_copy(den_s.at[pl.ds(nbase, NPW)],
                    dens.at[c].at[pl.ds(nbase, NPW)])

    def p2_issue(t, k, i, hp, wait_prev):
        def _waits():
            pltpu.make_async_copy(g[t], acc_s.at[db[t]], semc[t]).wait()

        if wait_prev is True:
            _waits()
        elif wait_prev is not False:
            pl.when(wait_prev)(_waits)

        @pl.loop(0, CH, step=16)
        def _(u):
            ia[t][pl.ds(u, 16)] = sdt0[k, 0, pl.ds(u, 16)] * HP + (fbase + hp)
            db[t][pl.ds(u, 16)] = sdt0[k, 1, pl.ds(u, 16)]
        pltpu.async_copy(featv.at[ia[t]], g[t], semg[t])
        pltpu.async_copy(exs_cs.at[i], exb[t], semx[t])

    def p2_finish(t, hp):
        pltpu.make_async_copy(featv.at[ia[t]], g[t], semg[t]).wait()
        pltpu.make_async_copy(exs_cs.at[0], exb[t], semx[t]).wait()

        @pl.loop(0, CH, step=2)
        def _(r):
            for rr in range(2):
                ri = r + rr
                av0 = plsc.load_gather(
                    exb[t], [jnp.full((16,), ri * 8 + 2 * hp, jnp.int32)])
                av1 = plsc.load_gather(
                    exb[t], [jnp.full((16,), ri * 8 + 2 * hp + 1, jnp.int32)])
                for j in range(4):
                    sl = pl.ds(j * 16, 16)
                    g[t][ri, sl] = g[t][ri, sl] * av0
                for j in range(4, 8):
                    sl = pl.ds(j * 16, 16)
                    g[t][ri, sl] = g[t][ri, sl] * av1

        pltpu.async_copy(g[t], acc_s.at[db[t]], semc[t], add=True)

    @pl.loop(0, HP)
    def _(hp):
        pltpu.sync_copy(zacc, acc_s.at[pl.ds(nbase, NPW)])
        plsc.subcore_barrier()

        pltpu.sync_copy(sd_cs.at[pl.ds(0, 5)], sdt0)
        p2_issue(0, 0, 0, hp, False)
        p2_issue(1, 1, 1, hp, False)

        @pl.loop(0, NCHUNK - 2, step=3)
        def _(i):
            pltpu.async_copy(sd_cs.at[pl.ds(i + 3, 5)], sdt1, semd1)
            p2_issue(2, 2, i + 2, hp, i > 0)
            p2_finish(0, hp)
            p2_issue(0, 3, i + 3, hp, True)
            p2_finish(1, hp)
            p2_issue(1, 4, i + 4, hp, True)
            p2_finish(2, hp)
            pltpu.make_async_copy(sd_cs.at[pl.ds(0, 5)], sdt1, semd1).wait()
            for kk in range(5):
                for dd in range(2):
                    @pl.loop(0, CH, step=16)
                    def _(u):
                        sdt0[kk, dd, pl.ds(u, 16)] = sdt1[kk, dd, pl.ds(u, 16)]

        p2_finish(0, hp)
        p2_finish(1, hp)

        for t in (0, 1, 2):
            pltpu.make_async_copy(g[t], acc_s.at[db[t]], semc[t]).wait()

        plsc.subcore_barrier()
        pltpu.sync_copy(acc_s.at[pl.ds(nbase, NPW)],
                        accs.at[c].at[pl.ds(nbase, NPW), pl.ds(hp * PD, PD)])
        plsc.subcore_barrier()


def _sc_gat(featv, eldv, erdv, sd, zacc, zden):
    mesh = plsc.VectorSubcoreMesh(core_axis_name="c", subcore_axis_name="s")
    cp = pltpu.CompilerParams()
    for fld, val in (("needs_layout_passes", False),
                     ("use_tc_tiling_on_sc", False)):
        if fld in pltpu.CompilerParams.__dataclass_fields__:
            cp = dataclasses.replace(cp, **{fld: val})
    kern = functools.partial(
        pl.kernel,
        compiler_params=cp,
        out_type=[
            jax.ShapeDtypeStruct((NC, NPAD, H * D), jnp.float32),
            jax.ShapeDtypeStruct((NC, NPAD, 2 * H), jnp.float32),
            jax.ShapeDtypeStruct((NC, NS, NCHUNK, 8 * CH), jnp.float32),
        ],
        mesh=mesh,
        scratch_types=[
            pltpu.VMEM_SHARED((NPAD, PD), jnp.float32),
            pltpu.VMEM_SHARED((NPAD, 2 * H), jnp.float32),
            pltpu.VMEM((5, 2, CH), jnp.int32),
            pltpu.VMEM((5, 2, CH), jnp.int32),
            pltpu.VMEM((CH,), jnp.int32),
            pltpu.VMEM((CH,), jnp.int32),
            pltpu.VMEM((CH,), jnp.int32),
            pltpu.VMEM((CH,), jnp.int32),
            pltpu.VMEM((CH,), jnp.int32),
            pltpu.VMEM((CH,), jnp.int32),
            pltpu.VMEM((CH,), jnp.int32),
            pltpu.VMEM((CH, 2 * H), jnp.float32),
            pltpu.VMEM((CH, 2 * H), jnp.float32),
            pltpu.VMEM((8 * CH + 16,), jnp.float32),
            pltpu.VMEM((8 * CH,), jnp.float32),
            pltpu.VMEM((8 * CH,), jnp.float32),
            pltpu.VMEM((8 * CH,), jnp.float32),
            pltpu.VMEM((CH, PD), jnp.float32),
            pltpu.VMEM((CH, PD), jnp.float32),
            pltpu.VMEM((CH, PD), jnp.float32),
            pltpu.SemaphoreType.DMA,
            pltpu.SemaphoreType.DMA,
            pltpu.SemaphoreType.DMA,
            pltpu.SemaphoreType.DMA,
            pltpu.SemaphoreType.DMA,
            pltpu.SemaphoreType.DMA,
            pltpu.SemaphoreType.DMA,
            pltpu.SemaphoreType.DMA,
            pltpu.SemaphoreType.DMA,
            pltpu.SemaphoreType.DMA,
            pltpu.SemaphoreType.DMA,
        ],
    )(_sc_gat_body)
    return kern(featv, eldv, erdv, sd, zacc, zden)


def _tc2a_body(acc0_ref, acc1_ref, den0_ref, den1_ref, Ws1_ref, bs1_ref,
               Ws2_ref, f0_ref, f1_ref, wsum_ref):
    i = pl.program_id(0)
    lane = lax.broadcasted_iota(jnp.int32, (1, 8), 1)

    @pl.when(i == 0)
    def _():
        wsum_ref[...] = jnp.zeros_like(wsum_ref)

    ts = []
    for acc_ref, den_ref, f_ref in [(acc0_ref, den0_ref, f0_ref),
                                    (acc1_ref, den1_ref, f1_ref)]:
        d = den_ref[...][0][:, :H]
        r = 1.0 / jnp.maximum(d, 1e-9)
        re = jnp.broadcast_to(r[:, :, None], (BLK, H, D)).reshape(BLK, H * D)
        x = acc_ref[...][0] * re
        f = jnp.where(x > 0, x, jnp.exp(jnp.minimum(x, 0.0)) - 1.0)
        f_ref[...] = f
        t = (jnp.tanh(jnp.dot(f, Ws1_ref[...],
                              preferred_element_type=jnp.float32)
                      + bs1_ref[...][None, :]) @ Ws2_ref[...]).sum()
        ts.append(t)

    wsum_ref[...] += (jnp.where(lane == 0, ts[0], 0.0)
                      + jnp.where(lane == 1, ts[1], 0.0))


def _tc2b_body(f0_ref, f1_ref, wsum_ref, out_ref):
    row = wsum_ref[...]
    w0 = row[0, 0] / N
    w1 = row[0, 1] / N
    m = jnp.maximum(w0, w1)
    b0 = jnp.exp(w0 - m)
    b1 = jnp.exp(w1 - m)
    s = b0 + b1
    out_ref[...] = (b0 / s) * f0_ref[...] + (b1 / s) * f1_ref[...]


def _tc2(accs, dens, Ws1, bs1, Ws2):
    f0, f1, wsum = pl.pallas_call(
        _tc2a_body,
        grid=(NB,),
        in_specs=[
            pl.BlockSpec((1, BLK, H * D), lambda i: (0, i, 0)),
            pl.BlockSpec((1, BLK, H * D), lambda i: (1, i, 0)),
            pl.BlockSpec((1, BLK, 2 * H), lambda i: (0, i, 0)),
            pl.BlockSpec((1, BLK, 2 * H), lambda i: (1, i, 0)),
            pl.BlockSpec((H * D, HID), lambda i: (0, 0)),
            pl.BlockSpec((HID,), lambda i: (0,)),
            pl.BlockSpec((HID, 1), lambda i: (0, 0)),
        ],
        out_specs=[
            pl.BlockSpec((BLK, H * D), lambda i: (i, 0)),
            pl.BlockSpec((BLK, H * D), lambda i: (i, 0)),
            pl.BlockSpec((1, 8), lambda i: (0, 0)),
        ],
        out_shape=[
            jax.ShapeDtypeStruct((N, H * D), jnp.float32),
            jax.ShapeDtypeStruct((N, H * D), jnp.float32),
            jax.ShapeDtypeStruct((1, 8), jnp.float32),
        ],
    )(accs, accs, dens, dens, Ws1, bs1, Ws2)
    out = pl.pallas_call(
        _tc2b_body,
        grid=(NB,),
        in_specs=[
            pl.BlockSpec((BLK, H * D), lambda i: (i, 0)),
            pl.BlockSpec((BLK, H * D), lambda i: (i, 0)),
            pl.BlockSpec((1, 8), lambda i: (0, 0)),
        ],
        out_specs=pl.BlockSpec((BLK, H * D), lambda i: (i, 0)),
        out_shape=jax.ShapeDtypeStruct((N, H * D), jnp.float32),
    )(f0, f1, wsum)
    return out


def kernel(h, edge_index_0, edge_index_1, W0, al0, ar0, W1, al1, ar1,
           Ws1, bs1, Ws2, layer_number):
    feat, eld, erd = _tc1(h, W0, al0, ar0, W1, al1, ar1)
    featv = feat.reshape(NC * N * HP, PD)
    eldv = eld.reshape(NC * N, 2 * H)
    erdv = erd.reshape(NC * N, 2 * H)
    srcs = jnp.stack([edge_index_0[0], edge_index_1[0]]).astype(jnp.int32)
    dsts = jnp.stack([edge_index_0[1], edge_index_1[1]]).astype(jnp.int32)
    sd = jnp.stack([srcs.reshape(NC, NS, NCHUNK, CH),
                    dsts.reshape(NC, NS, NCHUNK, CH)], axis=3)
    sd = jnp.pad(sd, ((0, 0), (0, 0), (0, 128 - NCHUNK), (0, 0), (0, 0)))
    zacc = jnp.zeros((NPW, PD), jnp.float32)
    zden = jnp.zeros((NPW, 2 * H), jnp.float32)
    accs, dens, _ = _sc_gat(featv, eldv, erdv, sd, zacc, zden)
    return _tc2(accs, dens, Ws1, bs1, Ws2)
